# Initial kernel scaffold; baseline (speedup 1.0000x reference)
import functools

import jax
import jax.numpy as jnp
from jax import lax
from jax.experimental import pallas as pl
from jax.experimental.pallas import tpu as pltpu
from jax.experimental.pallas import tpu_sc as plsc

N = 10000
NG = 2
NT = NG * N
E1 = 320000
E2 = NG * E1
PW = 16
CHUNK = 128
NCHUNKS = E2 // CHUNK
NCHUNKS_G = E1 // CHUNK
NC, NS = 2, 16
NW = NC * NS
R_E = 3200
GB_E = E2 // R_E
HB_E = GB_E // 2
NPB = 8
EPS = 1e-5
F32 = jnp.float32
_DIMS = [(64, 128), (128, 256), (256, 512)]

_sds = jax.ShapeDtypeStruct


def _relu(x):
    return jnp.maximum(x, 0.0)


def _scale_shift(st_row, g, b):
    m = st_row[0] * (1.0 / E1)
    v = st_row[1] * (1.0 / E1) - m * m
    sc = g * lax.rsqrt(v + EPS)
    return sc, b - m * sc


def _embed(x, win_s, bin_s):
    def body(x_ref, w_ref, b_ref, h_ref):
        h_ref[...] = x_ref[...][:, None] * w_ref[0, 0, :] + b_ref[0, 0, :]

    return pl.pallas_call(
        body,
        grid=(NG,),
        in_specs=[pl.BlockSpec((N,), lambda g: (g,)),
                  pl.BlockSpec((1, 8, 64), lambda g: (g, 0, 0)),
                  pl.BlockSpec((1, 8, 64), lambda g: (g, 0, 0))],
        out_specs=pl.BlockSpec((N, 64), lambda g: (g, 0)),
        out_shape=_sds((NT, 64), F32),
    )(x, win_s, bin_s)


def _node_pre(h, pos, w1a_s, w1b_s, b1_s, e):
    W_ = e + PW
    NB = N // NPB

    def body(h_ref, pos_ref, wa_ref, wb_ref, b_ref, td_ref, ts_ref):
        hv = h_ref[...]
        a = jnp.dot(hv, wa_ref[0], preferred_element_type=F32) + b_ref[0, 0, :]
        bb = jnp.dot(hv, wb_ref[0], preferred_element_type=F32)
        p3 = pos_ref[...][:, :3]
        z = jnp.zeros((NB, PW - 3), F32)
        td_ref[...] = jnp.concatenate([a, p3, z], axis=1)
        ts_ref[...] = jnp.concatenate([bb, -p3, z], axis=1)

    return pl.pallas_call(
        body,
        grid=(NG, NPB),
        in_specs=[pl.BlockSpec((NB, e), lambda g, j: (g * NPB + j, 0)),
                  pl.BlockSpec((NB, 8), lambda g, j: (g * NPB + j, 0)),
                  pl.BlockSpec((1, e, e), lambda g, j: (g, 0, 0)),
                  pl.BlockSpec((1, e, e), lambda g, j: (g, 0, 0)),
                  pl.BlockSpec((1, 8, e), lambda g, j: (g, 0, 0))],
        out_specs=[pl.BlockSpec((NB, W_), lambda g, j: (g * NPB + j, 0)),
                   pl.BlockSpec((NB, W_), lambda g, j: (g * NPB + j, 0))],
        out_shape=[_sds((NT, W_), F32), _sds((NT, W_), F32)],
    )(h, pos, w1a_s, w1b_s, b1_s)


def _edge_t1(GA, GB, w1c_s, e):
    W_ = e + PW

    def body(ga_ref, gb_ref, w_ref, y1_ref, pd_ref, st_ref):
        i = pl.program_id(0)
        s = ga_ref[...] + gb_ref[...]
        a = s[:, :e]
        pd = s[:, e:e + 3]
        d = jnp.sqrt(jnp.sum(pd * pd, axis=1, keepdims=True))
        y1 = a + d * w_ref[0, 0, :]
        y1_ref[...] = y1
        pd_ref[...] = jnp.concatenate(
            [pd, jnp.zeros((R_E, 5), F32)], axis=1)
        part = jnp.concatenate(
            [jnp.sum(y1, axis=0)[None], jnp.sum(y1 * y1, axis=0)[None],
             jnp.zeros((6, e), F32)], axis=0)[None]

        @pl.when(i % HB_E == 0)
        def _():
            st_ref[...] = part

        @pl.when(i % HB_E != 0)
        def _():
            st_ref[...] += part

    return pl.pallas_call(
        body,
        grid=(GB_E,),
        in_specs=[pl.BlockSpec((R_E, W_), lambda i: (i, 0)),
                  pl.BlockSpec((R_E, W_), lambda i: (i, 0)),
                  pl.BlockSpec((1, 8, e), lambda i: (i // HB_E, 0, 0))],
        out_specs=[pl.BlockSpec((R_E, e), lambda i: (i, 0)),
                   pl.BlockSpec((R_E, 8), lambda i: (i, 0)),
                   pl.BlockSpec((1, 8, e), lambda i: (i // HB_E, 0, 0))],
        out_shape=[_sds((E2, e), F32), _sds((E2, 8), F32),
                   _sds((NG, 8, e), F32)],
        compiler_params=pltpu.CompilerParams(
            dimension_semantics=("arbitrary",)),
    )(GA, GB, w1c_s)


def _edge_bn_mm(Y, st, g_s, b_s, W_s, bias_s, e, n_msg, cw):
    def body(y_ref, st_ref, g_ref, b_ref, w_ref, bias_ref, *outs):
        i = pl.program_id(0)
        y2_ref, st2_ref = outs[0], outs[1]
        msg_refs = outs[2:]
        sc, sh = _scale_shift(st_ref[0], g_ref[0, 0, :], b_ref[0, 0, :])
        z = _relu(y_ref[...] * sc + sh)
        for h in range(n_msg):
            msg_refs[h][...] = z[:, h * cw:(h + 1) * cw]
        y2 = jnp.dot(z, w_ref[0], preferred_element_type=F32) + bias_ref[0, 0, :]
        y2_ref[...] = y2
        part = jnp.concatenate(
            [jnp.sum(y2, axis=0)[None], jnp.sum(y2 * y2, axis=0)[None],
             jnp.zeros((6, e), F32)], axis=0)[None]

        @pl.when(i % HB_E == 0)
        def _():
            st2_ref[...] = part

        @pl.when(i % HB_E != 0)
        def _():
            st2_ref[...] += part

    out_specs = [pl.BlockSpec((R_E, e), lambda i: (i, 0)),
                 pl.BlockSpec((1, 8, e), lambda i: (i // HB_E, 0, 0))]
    out_shape = [_sds((E2, e), F32), _sds((NG, 8, e), F32)]
    for _ in range(n_msg):
        out_specs.append(pl.BlockSpec((R_E, cw), lambda i: (i, 0)))
        out_shape.append(_sds((E2, cw), F32))

    return pl.pallas_call(
        body,
        grid=(GB_E,),
        in_specs=[pl.BlockSpec((R_E, e), lambda i: (i, 0)),
                  pl.BlockSpec((1, 8, e), lambda i: (i // HB_E, 0, 0)),
                  pl.BlockSpec((1, 8, e), lambda i: (i // HB_E, 0, 0)),
                  pl.BlockSpec((1, 8, e), lambda i: (i // HB_E, 0, 0)),
                  pl.BlockSpec((1, e, e), lambda i: (i // HB_E, 0, 0)),
                  pl.BlockSpec((1, 8, e), lambda i: (i // HB_E, 0, 0))],
        out_specs=out_specs,
        out_shape=out_shape,
        compiler_params=pltpu.CompilerParams(
            dimension_semantics=("arbitrary",)),
    )(Y, st, g_s, b_s, W_s, bias_s)


def _edge_t4(Y3, st, g_s, b_s, wp2_s, bp2_s, PD, e):
    def body(y_ref, st_ref, g_ref, b_ref, w_ref, bias_ref, pd_ref, wp_ref):
        sc, sh = _scale_shift(st_ref[0], g_ref[0, 0, :], b_ref[0, 0, :])
        z = _relu(y_ref[...] * sc + sh)
        we = jnp.sum(z * w_ref[0, 0, :], axis=1, keepdims=True) \
            + bias_ref[0, 0, 0]
        wp = pd_ref[...][:, :3] * we
        wp_ref[...] = jnp.concatenate(
            [wp, jnp.ones((R_E, 1), F32), jnp.zeros((R_E, 4), F32)], axis=1)

    return pl.pallas_call(
        body,
        grid=(GB_E,),
        in_specs=[pl.BlockSpec((R_E, e), lambda i: (i, 0)),
                  pl.BlockSpec((1, 8, e), lambda i: (i // HB_E, 0, 0)),
                  pl.BlockSpec((1, 8, e), lambda i: (i // HB_E, 0, 0)),
                  pl.BlockSpec((1, 8, e), lambda i: (i // HB_E, 0, 0)),
                  pl.BlockSpec((1, 8, e), lambda i: (i // HB_E, 0, 0)),
                  pl.BlockSpec((1, 8, 8), lambda i: (i // HB_E, 0, 0)),
                  pl.BlockSpec((R_E, 8), lambda i: (i, 0))],
        out_specs=pl.BlockSpec((R_E, 8), lambda i: (i, 0)),
        out_shape=_sds((E2, 8), F32),
    )(Y3, st, g_s, b_s, wp2_s, bp2_s, PD)


def _node_update(h, Msum, Psum, pos, wa_s, wb_s, bu1_s, g1_s, b1_s,
                 wu2_s, bu2_s, g2_s, b2_s, wo_s, bo_s, e, eo):
    def body(h_ref, ms_ref, ps_ref, pos_ref, wa, wb, bu1, g1, b1,
             wu2, bu2, g2, b2, wo, bo, hn_ref, pn_ref):
        ps = ps_ref[...]
        cnt = jnp.maximum(ps[:, 3:4], 1.0)
        ma = ms_ref[...] / cnt
        u = jnp.dot(h_ref[...], wa[0], preferred_element_type=F32) \
            + jnp.dot(ma, wb[0], preferred_element_type=F32) + bu1[0, 0, :]
        m = jnp.mean(u, axis=0)
        v = jnp.mean(u * u, axis=0) - m * m
        sc = g1[0, 0, :] * lax.rsqrt(v + EPS)
        z = _relu(u * sc + (b1[0, 0, :] - m * sc))
        u2 = jnp.dot(z, wu2[0], preferred_element_type=F32) + bu2[0, 0, :]
        m2 = jnp.mean(u2, axis=0)
        v2 = jnp.mean(u2 * u2, axis=0) - m2 * m2
        sc2 = g2[0, 0, :] * lax.rsqrt(v2 + EPS)
        z2 = _relu(u2 * sc2 + (b2[0, 0, :] - m2 * sc2))
        hn_ref[...] = jnp.dot(z2, wo[0], preferred_element_type=F32) \
            + bo[0, 0, :]
        pn_ref[...] = pos_ref[...] + jnp.concatenate(
            [ps[:, :3] / cnt, jnp.zeros((N, 5), F32)], axis=1)

    def vec(g):
        return pl.BlockSpec((1, 8, g), lambda i: (i, 0, 0))

    def mat(a, b):
        return pl.BlockSpec((1, a, b), lambda i: (i, 0, 0))

    return pl.pallas_call(
        body,
        grid=(NG,),
        in_specs=[pl.BlockSpec((N, e), lambda i: (i, 0)),
                  pl.BlockSpec((N, e), lambda i: (i, 0)),
                  pl.BlockSpec((N, 8), lambda i: (i, 0)),
                  pl.BlockSpec((N, 8), lambda i: (i, 0)),
                  mat(e, e), mat(e, e), vec(e), vec(e), vec(e),
                  mat(e, e), vec(e), vec(e), vec(e),
                  mat(e, eo), vec(eo)],
        out_specs=[pl.BlockSpec((N, eo), lambda i: (i, 0)),
                   pl.BlockSpec((N, 8), lambda i: (i, 0))],
        out_shape=[_sds((NT, eo), F32), _sds((NT, 8), F32)],
    )(h, Msum, Psum, pos, wa_s, wb_s, bu1_s, g1_s, b1_s,
      wu2_s, bu2_s, g2_s, b2_s, wo_s, bo_s)


def _pos_finish(pos, Psum):
    def body(pos_ref, ps_ref, pn_ref):
        ps = ps_ref[...]
        cnt = jnp.maximum(ps[:, 3:4], 1.0)
        pn_ref[...] = pos_ref[...] + jnp.concatenate(
            [ps[:, :3] / cnt, jnp.zeros((NT, 5), F32)], axis=1)

    return pl.pallas_call(
        body,
        out_shape=_sds((NT, 8), F32),
    )(pos, Psum)


def _sc_gather(Tdst, Tsrc, gidx, W_):
    niter = -(-NCHUNKS // NW)
    mesh = plsc.VectorSubcoreMesh(core_axis_name="c", subcore_axis_name="s")

    @functools.partial(
        pl.kernel,
        out_type=(_sds((E2, W_), F32), _sds((E2, W_), F32)),
        mesh=mesh,
        scratch_types=[pltpu.VMEM((CHUNK,), jnp.int32),
                       pltpu.VMEM((CHUNK,), jnp.int32),
                       pltpu.VMEM((CHUNK, W_), F32),
                       pltpu.VMEM((CHUNK, W_), F32),
                       pltpu.SemaphoreType.DMA,
                       pltpu.SemaphoreType.DMA],
    )
    def k(tdst, tsrc, gix, ga, gb, idxd, idxs, bufd, bufs, semd, sems):
        wid = lax.axis_index("s") * NC + lax.axis_index("c")

        def body(kk, carry):
            chunk = wid + NW * kk

            @pl.when(chunk < NCHUNKS)
            def _():
                base = chunk * CHUNK
                pltpu.sync_copy(gix.at[0, chunk], idxd)
                pltpu.sync_copy(gix.at[1, chunk], idxs)
                cd = pltpu.async_copy(tdst.at[idxd], bufd, semd)
                cs = pltpu.async_copy(tsrc.at[idxs], bufs, sems)
                cd.wait()
                cs.wait()
                pltpu.sync_copy(bufd, ga.at[pl.ds(base, CHUNK)])
                pltpu.sync_copy(bufs, gb.at[pl.ds(base, CHUNK)])

            return carry

        lax.fori_loop(0, niter, body, 0)

    return k(Tdst, Tsrc, gidx)


def _sc_scatter(msgs, WP, sidx, zeros_m, zeros_p, cw, nh):
    nrows = N // NS
    niter = -(-NCHUNKS_G // NS)
    mesh = plsc.VectorSubcoreMesh(core_axis_name="c", subcore_axis_name="s")

    out_type = []
    if nh:
        out_type.append(_sds((nh, NT, cw), F32))
    out_type.append(_sds((NT, 8), F32))

    scratch = [pltpu.VMEM((CHUNK,), jnp.int32),
               pltpu.VMEM((CHUNK, 8), F32),
               pltpu.VMEM_SHARED((N, 8), F32)]
    if nh:
        scratch += [pltpu.VMEM((CHUNK, cw), F32),
                    pltpu.VMEM_SHARED((N, cw), F32)]

    @functools.partial(
        pl.kernel,
        out_type=tuple(out_type),
        mesh=mesh,
        scratch_types=scratch,
    )
    def k(*args):
        a = list(args)
        msg_refs = a[:nh]
        wp, six, zm, zp = a[nh], a[nh + 1], a[nh + 2], a[nh + 3]
        pos_arg = nh + 4
        if nh:
            msum = a[pos_arg]
            psum = a[pos_arg + 1]
            idxv, bufp, accp, bufm, acc = a[pos_arg + 2:]
        else:
            psum = a[pos_arg]
            idxv, bufp, accp = a[pos_arg + 1:]
        c = lax.axis_index("c")
        s = lax.axis_index("s")
        row0 = s * nrows

        for h in range(nh):
            pltpu.sync_copy(zm.at[pl.ds(row0, nrows)],
                            acc.at[pl.ds(row0, nrows)])
            plsc.subcore_barrier()

            def mbody(kk, carry, _mref=msg_refs[h]):
                chunk = s + NS * kk

                @pl.when(chunk < NCHUNKS_G)
                def _():
                    gbase = (c * NCHUNKS_G + chunk) * CHUNK
                    pltpu.sync_copy(six.at[c, chunk], idxv)
                    pltpu.sync_copy(_mref.at[pl.ds(gbase, CHUNK)], bufm)
                    pltpu.sync_copy(bufm, acc.at[idxv], add=True)

                return carry

            lax.fori_loop(0, niter, mbody, 0)
            plsc.subcore_barrier()
            pltpu.sync_copy(acc.at[pl.ds(row0, nrows)],
                            msum.at[h, pl.ds(c * N + row0, nrows)])
            plsc.subcore_barrier()

        pltpu.sync_copy(zp.at[pl.ds(row0, nrows)],
                        accp.at[pl.ds(row0, nrows)])
        plsc.subcore_barrier()

        def pbody(kk, carry):
            chunk = s + NS * kk

            @pl.when(chunk < NCHUNKS_G)
            def _():
                gbase = (c * NCHUNKS_G + chunk) * CHUNK
                pltpu.sync_copy(six.at[c, chunk], idxv)
                pltpu.sync_copy(wp.at[pl.ds(gbase, CHUNK)], bufp)
                pltpu.sync_copy(bufp, accp.at[idxv], add=True)

            return carry

        lax.fori_loop(0, niter, pbody, 0)
        plsc.subcore_barrier()
        pltpu.sync_copy(accp.at[pl.ds(row0, nrows)],
                        psum.at[pl.ds(c * N + row0, nrows)])

    return k(*msgs, WP, sidx, zeros_m, zeros_p)


def _vec8(a, b):
    v = jnp.stack([a, b])
    out = jnp.zeros((2, 8, v.shape[1]), F32)
    return out.at[:, 0, :].set(v)


def _stack_layer(pr, pli, e):
    s = {}
    w1r, w1l = pr["msg1"]["W"], pli["msg1"]["W"]
    s["w1a"] = jnp.stack([w1r[:e], w1l[:e]])
    s["w1b"] = jnp.stack([w1r[e:2 * e], w1l[e:2 * e]])
    s["w1c"] = _vec8(w1r[2 * e], w1l[2 * e])
    s["b1"] = _vec8(pr["msg1"]["b"], pli["msg1"]["b"])
    s["g1"] = _vec8(pr["msg_bn1"]["g"], pli["msg_bn1"]["g"])
    s["bb1"] = _vec8(pr["msg_bn1"]["b"], pli["msg_bn1"]["b"])
    s["w2"] = jnp.stack([pr["msg2"]["W"], pli["msg2"]["W"]])
    s["b2"] = _vec8(pr["msg2"]["b"], pli["msg2"]["b"])
    s["g2"] = _vec8(pr["msg_bn2"]["g"], pli["msg_bn2"]["g"])
    s["bb2"] = _vec8(pr["msg_bn2"]["b"], pli["msg_bn2"]["b"])
    s["wp1"] = jnp.stack([pr["pos1"]["W"], pli["pos1"]["W"]])
    s["bp1"] = _vec8(pr["pos1"]["b"], pli["pos1"]["b"])
    s["gp"] = _vec8(pr["pos_bn"]["g"], pli["pos_bn"]["g"])
    s["bbp"] = _vec8(pr["pos_bn"]["b"], pli["pos_bn"]["b"])
    s["wp2"] = _vec8(pr["pos2"]["W"][:, 0], pli["pos2"]["W"][:, 0])
    bp2 = jnp.stack([pr["pos2"]["b"], pli["pos2"]["b"]])
    s["bp2"] = jnp.zeros((2, 8, 8), F32).at[:, 0, 0].set(bp2[:, 0])
    wur, wul = pr["upd1"]["W"], pli["upd1"]["W"]
    s["wua"] = jnp.stack([wur[:e], wul[:e]])
    s["wub"] = jnp.stack([wur[e:], wul[e:]])
    s["bu1"] = _vec8(pr["upd1"]["b"], pli["upd1"]["b"])
    s["gu1"] = _vec8(pr["upd_bn1"]["g"], pli["upd_bn1"]["g"])
    s["bu1b"] = _vec8(pr["upd_bn1"]["b"], pli["upd_bn1"]["b"])
    s["wu2"] = jnp.stack([pr["upd2"]["W"], pli["upd2"]["W"]])
    s["bu2"] = _vec8(pr["upd2"]["b"], pli["upd2"]["b"])
    s["gu2"] = _vec8(pr["upd_bn2"]["g"], pli["upd_bn2"]["g"])
    s["bu2b"] = _vec8(pr["upd_bn2"]["b"], pli["upd_bn2"]["b"])
    s["wo"] = jnp.stack([pr["out"]["W"], pli["out"]["W"]])
    s["bo"] = _vec8(pr["out"]["b"], pli["out"]["b"])
    return s


def kernel(rec_x, rec_pos, rec_edge_index, lig_x, lig_pos, lig_edge_index,
           params):
    x = jnp.concatenate([rec_x, lig_x], axis=0).reshape(NT)
    pos = jnp.concatenate(
        [jnp.concatenate([rec_pos, lig_pos], axis=0),
         jnp.zeros((NT, 5), F32)], axis=1)

    dst = jnp.concatenate([rec_edge_index[1], lig_edge_index[1] + N])
    src = jnp.concatenate([rec_edge_index[0], lig_edge_index[0] + N])
    gidx = jnp.stack([dst, src]).reshape(2, NCHUNKS, CHUNK)
    sidx = jnp.stack([rec_edge_index[1], lig_edge_index[1]]).reshape(
        2, NCHUNKS_G, CHUNK)

    win_s = _vec8(params["lin_in_rec"]["W"][0], params["lin_in_lig"]["W"][0])
    bin_s = _vec8(params["lin_in_rec"]["b"], params["lin_in_lig"]["b"])
    h = _embed(x, win_s, bin_s)

    for li, (e, eo) in enumerate(_DIMS):
        lp = _stack_layer(params["rec_layers"][li], params["lig_layers"][li],
                          e)
        last = li == len(_DIMS) - 1
        nh = 0 if last else -(-e // 128)
        cw = min(e, 128)

        Tdst, Tsrc = _node_pre(h, pos, lp["w1a"], lp["w1b"], lp["b1"], e)
        GA, GB = _sc_gather(Tdst, Tsrc, gidx, e + PW)
        Y1, PD, st1 = _edge_t1(GA, GB, lp["w1c"], e)
        Y2, st2 = _edge_bn_mm(Y1, st1, lp["g1"], lp["bb1"], lp["w2"],
                              lp["b2"], e, 0, cw)
        r3 = _edge_bn_mm(Y2, st2, lp["g2"], lp["bb2"], lp["wp1"],
                         lp["bp1"], e, nh, cw)
        Y3, st3 = r3[0], r3[1]
        msgs = r3[2:]
        WP = _edge_t4(Y3, st3, lp["gp"], lp["bbp"], lp["wp2"], lp["bp2"],
                      PD, e)

        zeros_m = jnp.zeros((N, cw), F32)
        zeros_p = jnp.zeros((N, 8), F32)
        sc_out = _sc_scatter(msgs, WP, sidx, zeros_m, zeros_p, cw, nh)
        if last:
            Psum = sc_out if not isinstance(sc_out, (tuple, list)) else sc_out[-1]
            pos = _pos_finish(pos, Psum)
        else:
            Msum_s, Psum = sc_out
            Msum = jnp.concatenate([Msum_s[i] for i in range(nh)], axis=1)
            h, pos = _node_update(
                h, Msum, Psum, pos, lp["wua"], lp["wub"], lp["bu1"],
                lp["gu1"], lp["bu1b"], lp["wu2"], lp["bu2"], lp["gu2"],
                lp["bu2b"], lp["wo"], lp["bo"], e, eo)

    return (pos[:N, :3], pos[N:, :3])


# speedup vs baseline: 2.2450x; 2.2450x over previous
import functools

import jax
import jax.numpy as jnp
from jax import lax
from jax.experimental import pallas as pl
from jax.experimental.pallas import tpu as pltpu
from jax.experimental.pallas import tpu_sc as plsc

N = 10000
NG = 2
NT = NG * N
E1 = 320000
E2 = NG * E1
CHUNK = 128
NCHUNKS = E2 // CHUNK
NCHUNKS_G = E1 // CHUNK
NC, NS = 2, 16
NW = NC * NS
NP_ = 10240
NRS = NP_ // NS
R_E = 3200
GB_E = E2 // R_E
HB_E = GB_E // 2
R_N = 2000
GB_N = NT // R_N
HB_N = GB_N // 2
NPB = 10
EPS = 1e-5
F32 = jnp.float32
_DIMS = [(64, 128), (128, 256), (256, 512)]

_sds = jax.ShapeDtypeStruct


BF16 = jnp.bfloat16


def _relu(x):
    return jnp.maximum(x, 0.0)


def _mm(x, w):
    return jnp.dot(x, w, preferred_element_type=F32)


def _tw(e):
    return -(-(e + 4) // 128) * 128


def _scale_shift(st_row, g, b, denom):
    m = st_row[0] * (1.0 / denom)
    v = st_row[1] * (1.0 / denom)
    sc = g / jnp.sqrt(v + EPS)
    return sc, b - m * sc


def _stats_update(st_ref, y, e, r, i, hb):
    bs = jnp.sum(y, axis=0)
    bm = bs * (1.0 / r)
    yc = y - bm
    bm2 = jnp.sum(yc * yc, axis=0)
    zpad = jnp.zeros((6, e), F32)

    @pl.when(i % hb == 0)
    def _():
        st_ref[...] = jnp.concatenate(
            [bs[None], bm2[None], zpad], axis=0)[None]

    @pl.when(i % hb != 0)
    def _():
        cur = st_ref[...]
        na = (i % hb).astype(F32) * r
        ma = cur[0, 0, :] * (1.0 / na)
        delta = bm - ma
        corr = delta * delta * (na * r / (na + r))
        st_ref[...] = jnp.concatenate(
            [(cur[0, 0, :] + bs)[None],
             (cur[0, 1, :] + bm2 + corr)[None], zpad], axis=0)[None]


def _embed(x, win_s, bin_s):
    def body(x_ref, w_ref, b_ref, h_ref):
        h_ref[...] = x_ref[...] * w_ref[0, 0, :] + b_ref[0, 0, :]

    return pl.pallas_call(
        body,
        grid=(GB_N,),
        in_specs=[pl.BlockSpec((R_N, 1), lambda g: (g, 0)),
                  pl.BlockSpec((1, 8, 64), lambda g: (g // HB_N, 0, 0)),
                  pl.BlockSpec((1, 8, 64), lambda g: (g // HB_N, 0, 0))],
        out_specs=pl.BlockSpec((R_N, 64), lambda g: (g, 0)),
        out_shape=_sds((NT, 64), F32),
    )(x, win_s, bin_s)


def _node_pre(h, pos, w1a_s, w1b_s, b1_s, e):
    W_ = _tw(e)
    NB = N // NPB

    def body(h_ref, pos_ref, wa_ref, wb_ref, b_ref, td_ref, ts_ref):
        hv = h_ref[...]
        a = _mm(hv, wa_ref[0]) + b_ref[0, 0, :]
        bb = _mm(hv, wb_ref[0])
        p3 = pos_ref[...][:, :3]
        z = jnp.zeros((NB, W_ - e - 3), F32)
        td_ref[...] = jnp.concatenate([a, p3, z], axis=1)
        ts_ref[...] = jnp.concatenate([bb, -p3, z], axis=1)

    return pl.pallas_call(
        body,
        grid=(NG, NPB),
        in_specs=[pl.BlockSpec((NB, e), lambda g, j: (g * NPB + j, 0)),
                  pl.BlockSpec((NB, 8), lambda g, j: (g * NPB + j, 0)),
                  pl.BlockSpec((1, e, e), lambda g, j: (g, 0, 0)),
                  pl.BlockSpec((1, e, e), lambda g, j: (g, 0, 0)),
                  pl.BlockSpec((1, 8, e), lambda g, j: (g, 0, 0))],
        out_specs=[pl.BlockSpec((NB, W_), lambda g, j: (g * NPB + j, 0)),
                   pl.BlockSpec((NB, W_), lambda g, j: (g * NPB + j, 0))],
        out_shape=[_sds((NT, W_), F32), _sds((NT, W_), F32)],
    )(h, pos, w1a_s, w1b_s, b1_s)


def _edge_t1(GA, GB, w1c_s, e):
    W_ = _tw(e)

    def body(ga_ref, gb_ref, w_ref, y1_ref, pd_ref, st_ref):
        i = pl.program_id(0)
        s = ga_ref[...] + gb_ref[...]
        a = s[:, :e]
        pd = s[:, e:e + 3]
        d = jnp.sqrt(jnp.sum(pd * pd, axis=1, keepdims=True))
        y1 = a + d * w_ref[0, 0, :]
        y1_ref[...] = y1
        pd_ref[...] = jnp.concatenate(
            [pd, jnp.zeros((R_E, 5), F32)], axis=1)
        _stats_update(st_ref, y1, e, R_E, i, HB_E)

    return pl.pallas_call(
        body,
        grid=(GB_E,),
        in_specs=[pl.BlockSpec((R_E, W_), lambda i: (i, 0)),
                  pl.BlockSpec((R_E, W_), lambda i: (i, 0)),
                  pl.BlockSpec((1, 8, e), lambda i: (i // HB_E, 0, 0))],
        out_specs=[pl.BlockSpec((R_E, e), lambda i: (i, 0)),
                   pl.BlockSpec((R_E, 8), lambda i: (i, 0)),
                   pl.BlockSpec((1, 8, e), lambda i: (i // HB_E, 0, 0))],
        out_shape=[_sds((E2, e), F32), _sds((E2, 8), F32),
                   _sds((NG, 8, e), F32)],
        compiler_params=pltpu.CompilerParams(
            dimension_semantics=("arbitrary",)),
    )(GA, GB, w1c_s)


def _bn_mm(Y, st, g_s, b_s, W_s, bias_s, din, dout, R, HB, denom,
           n_msg=0, cw=0, emit_stats=True):
    nrtot = Y.shape[0]

    def body(y_ref, st_ref, g_ref, b_ref, w_ref, bias_ref, *outs):
        i = pl.program_id(0)
        y2_ref = outs[0]
        msg_refs = outs[2:] if emit_stats else outs[1:]
        sc, sh = _scale_shift(st_ref[0], g_ref[0, 0, :], b_ref[0, 0, :],
                              denom)
        z = _relu(y_ref[...] * sc + sh)
        for h in range(n_msg):
            msg_refs[h][...] = z[:, h * cw:(h + 1) * cw]
        y2 = _mm(z, w_ref[0]) + bias_ref[0, 0, :]
        y2_ref[...] = y2
        if emit_stats:
            _stats_update(outs[1], y2, dout, R, i, HB)

    out_specs = [pl.BlockSpec((R, dout), lambda i: (i, 0))]
    out_shape = [_sds((nrtot, dout), F32)]
    if emit_stats:
        out_specs.append(pl.BlockSpec((1, 8, dout),
                                      lambda i: (i // HB, 0, 0)))
        out_shape.append(_sds((NG, 8, dout), F32))
    for _ in range(n_msg):
        out_specs.append(pl.BlockSpec((R, cw), lambda i: (i, 0)))
        out_shape.append(_sds((nrtot, cw), F32))

    return pl.pallas_call(
        body,
        grid=(nrtot // R,),
        in_specs=[pl.BlockSpec((R, din), lambda i: (i, 0)),
                  pl.BlockSpec((1, 8, din), lambda i: (i // HB, 0, 0)),
                  pl.BlockSpec((1, 8, din), lambda i: (i // HB, 0, 0)),
                  pl.BlockSpec((1, 8, din), lambda i: (i // HB, 0, 0)),
                  pl.BlockSpec((1, din, dout), lambda i: (i // HB, 0, 0)),
                  pl.BlockSpec((1, 8, dout), lambda i: (i // HB, 0, 0))],
        out_specs=out_specs,
        out_shape=out_shape,
        compiler_params=pltpu.CompilerParams(
            dimension_semantics=("arbitrary",)),
    )(Y, st, g_s, b_s, W_s, bias_s)


def _edge_t4(Y3, st3, gp_s, bp_s, wp2_s, bp2_s, PD, e, last,
             Y2=None, st2=None, g2_s=None, b2_s=None):
    off = 0 if last else e
    ngr = -(-(off + 4) // 128)

    def body(*refs):
        if last:
            (y3_ref, st_ref, g_ref, b_ref, w_ref, bias_ref, pd_ref) = \
                refs[:7]
            grp_refs = refs[7:]
        else:
            (y3_ref, st_ref, g_ref, b_ref, w_ref, bias_ref, pd_ref,
             y2_ref, st2_ref, g2_ref, b2_ref) = refs[:11]
            grp_refs = refs[11:]
        sc, sh = _scale_shift(st_ref[0], g_ref[0, 0, :], b_ref[0, 0, :], E1)
        z = _relu(y3_ref[...] * sc + sh)
        we = _mm(z, w_ref[0, 0, :][:, None]) + bias_ref[0, 0, 0]
        wp = pd_ref[...][:, :3] * we
        parts = []
        if not last:
            sc2, sh2 = _scale_shift(st2_ref[0], g2_ref[0, 0, :],
                                    b2_ref[0, 0, :], E1)
            parts.append(_relu(y2_ref[...] * sc2 + sh2))
        parts += [wp, jnp.ones((R_E, 1), F32),
                  jnp.zeros((R_E, ngr * 128 - off - 4), F32)]
        pay = jnp.concatenate(parts, axis=1)
        for gi in range(ngr):
            grp_refs[gi][...] = pay[:, gi * 128:(gi + 1) * 128]

    def vec(d):
        return pl.BlockSpec((1, 8, d), lambda i: (i // HB_E, 0, 0))

    in_specs = [pl.BlockSpec((R_E, e), lambda i: (i, 0)),
                vec(e), vec(e), vec(e), vec(e), vec(8),
                pl.BlockSpec((R_E, 8), lambda i: (i, 0))]
    ins = [Y3, st3, gp_s, bp_s, wp2_s, bp2_s, PD]
    if not last:
        in_specs += [pl.BlockSpec((R_E, e), lambda i: (i, 0)),
                     vec(e), vec(e), vec(e)]
        ins += [Y2, st2, g2_s, b2_s]

    return pl.pallas_call(
        body,
        grid=(GB_E,),
        in_specs=in_specs,
        out_specs=[pl.BlockSpec((R_E, 128), lambda i: (i, 0))
                   for _ in range(ngr)],
        out_shape=[_sds((E2, 128), F32) for _ in range(ngr)],
    )(*ins)


def _aggr_finish(Psum, pos, Msum=None):
    with_msg = Msum is not None
    e = Msum.shape[1] if with_msg else 0

    def body(*refs):
        if with_msg:
            ps_ref, pos_ref, ms_ref, pn_ref, ma_ref = refs
        else:
            ps_ref, pos_ref, pn_ref = refs
        ps = ps_ref[...]
        cnt = jnp.maximum(ps[:, 3:4], 1.0)
        pn_ref[...] = pos_ref[...] + jnp.concatenate(
            [ps[:, :3] / cnt, jnp.zeros((R_N, 5), F32)], axis=1)
        if with_msg:
            ma_ref[...] = ms_ref[...] / cnt

    in_specs = [pl.BlockSpec((R_N, 8), lambda i: (i, 0)),
                pl.BlockSpec((R_N, 8), lambda i: (i, 0))]
    ins = [Psum, pos]
    out_specs = [pl.BlockSpec((R_N, 8), lambda i: (i, 0))]
    out_shape = [_sds((NT, 8), F32)]
    if with_msg:
        in_specs.append(pl.BlockSpec((R_N, e), lambda i: (i, 0)))
        ins.append(Msum)
        out_specs.append(pl.BlockSpec((R_N, e), lambda i: (i, 0)))
        out_shape.append(_sds((NT, e), F32))

    return pl.pallas_call(
        body,
        grid=(GB_N,),
        in_specs=in_specs,
        out_specs=out_specs,
        out_shape=out_shape,
    )(*ins)


def _nu1(h, Mavg, wa_s, wb_s, b_s, e):
    def body(h_ref, ma_ref, wa_ref, wb_ref, b_ref, u_ref, st_ref):
        i = pl.program_id(0)
        u = _mm(h_ref[...], wa_ref[0]) + _mm(ma_ref[...], wb_ref[0]) \
            + b_ref[0, 0, :]
        u_ref[...] = u
        _stats_update(st_ref, u, e, R_N, i, HB_N)

    return pl.pallas_call(
        body,
        grid=(GB_N,),
        in_specs=[pl.BlockSpec((R_N, e), lambda i: (i, 0)),
                  pl.BlockSpec((R_N, e), lambda i: (i, 0)),
                  pl.BlockSpec((1, e, e), lambda i: (i // HB_N, 0, 0)),
                  pl.BlockSpec((1, e, e), lambda i: (i // HB_N, 0, 0)),
                  pl.BlockSpec((1, 8, e), lambda i: (i // HB_N, 0, 0))],
        out_specs=[pl.BlockSpec((R_N, e), lambda i: (i, 0)),
                   pl.BlockSpec((1, 8, e), lambda i: (i // HB_N, 0, 0))],
        out_shape=[_sds((NT, e), F32), _sds((NG, 8, e), F32)],
        compiler_params=pltpu.CompilerParams(
            dimension_semantics=("arbitrary",)),
    )(h, Mavg, wa_s, wb_s, b_s)


def _sc_gather(Tdst, Tsrc, gidx, W_):
    niter = -(-NCHUNKS // NW)
    mesh = plsc.VectorSubcoreMesh(core_axis_name="c", subcore_axis_name="s")

    @functools.partial(
        pl.kernel,
        out_type=(_sds((E2, W_), F32), _sds((E2, W_), F32)),
        mesh=mesh,
        scratch_types=[pltpu.VMEM((CHUNK,), jnp.int32),
                       pltpu.VMEM((CHUNK,), jnp.int32),
                       pltpu.VMEM((CHUNK, W_), F32),
                       pltpu.VMEM((CHUNK, W_), F32),
                       pltpu.SemaphoreType.DMA,
                       pltpu.SemaphoreType.DMA],
    )
    def k(tdst, tsrc, gix, ga, gb, idxd, idxs, bufd, bufs, semd, sems):
        wid = lax.axis_index("s") * NC + lax.axis_index("c")

        def body(kk, carry):
            chunk = wid + NW * kk

            @pl.when(chunk < NCHUNKS)
            def _():
                base = chunk * CHUNK
                pltpu.sync_copy(gix.at[0, chunk], idxd)
                pltpu.sync_copy(gix.at[1, chunk], idxs)
                cd = pltpu.async_copy(tdst.at[idxd], bufd, semd)
                cs = pltpu.async_copy(tsrc.at[idxs], bufs, sems)
                cd.wait()
                cs.wait()
                pltpu.sync_copy(bufd, ga.at[pl.ds(base, CHUNK)])
                pltpu.sync_copy(bufs, gb.at[pl.ds(base, CHUNK)])

            return carry

        lax.fori_loop(0, niter, body, 0)

    return k(Tdst, Tsrc, gidx)


def _sc_scatter(groups, sidx, zeros128, ngr):
    niter = -(-NCHUNKS_G // NS)
    mesh = plsc.VectorSubcoreMesh(core_axis_name="c", subcore_axis_name="s")

    @functools.partial(
        pl.kernel,
        out_type=_sds((ngr, NG * NP_, 128), F32),
        mesh=mesh,
        scratch_types=[pltpu.VMEM((CHUNK,), jnp.int32),
                       pltpu.VMEM((CHUNK, 128), F32),
                       pltpu.VMEM_SHARED((NP_, 128), F32)],
    )
    def k(*args):
        grp_refs = args[:ngr]
        six, zm, gsum = args[ngr], args[ngr + 1], args[ngr + 2]
        idxv, bufm, acc = args[ngr + 3:]
        c = lax.axis_index("c")
        s = lax.axis_index("s")
        row0 = pl.multiple_of(s * NRS, 8)
        out0 = pl.multiple_of(c * NP_ + row0, 8)

        for h in range(ngr):
            pltpu.sync_copy(zm.at[pl.ds(row0, NRS)],
                            acc.at[pl.ds(row0, NRS)])
            plsc.subcore_barrier()

            def mbody(kk, carry, _mref=grp_refs[h]):
                chunk = s + NS * kk

                @pl.when(chunk < NCHUNKS_G)
                def _():
                    gbase = (c * NCHUNKS_G + chunk) * CHUNK
                    pltpu.sync_copy(six.at[c, chunk], idxv)
                    pltpu.sync_copy(_mref.at[pl.ds(gbase, CHUNK)], bufm)
                    pltpu.sync_copy(bufm, acc.at[idxv], add=True)

                return carry

            lax.fori_loop(0, niter, mbody, 0)
            plsc.subcore_barrier()
            pltpu.sync_copy(acc.at[pl.ds(row0, NRS)],
                            gsum.at[h, pl.ds(out0, NRS)])
            plsc.subcore_barrier()

    return k(*groups, sidx, zeros128)


def _vec8(a, b):
    v = jnp.stack([a, b])
    out = jnp.zeros((2, 8, v.shape[1]), F32)
    return out.at[:, 0, :].set(v)


def _stack_layer(pr, pli, e):
    s = {}
    w1r, w1l = pr["msg1"]["W"], pli["msg1"]["W"]
    s["w1a"] = jnp.stack([w1r[:e], w1l[:e]])
    s["w1b"] = jnp.stack([w1r[e:2 * e], w1l[e:2 * e]])
    s["w1c"] = _vec8(w1r[2 * e], w1l[2 * e])
    s["b1"] = _vec8(pr["msg1"]["b"], pli["msg1"]["b"])
    s["g1"] = _vec8(pr["msg_bn1"]["g"], pli["msg_bn1"]["g"])
    s["bb1"] = _vec8(pr["msg_bn1"]["b"], pli["msg_bn1"]["b"])
    s["w2"] = jnp.stack([pr["msg2"]["W"], pli["msg2"]["W"]])
    s["b2"] = _vec8(pr["msg2"]["b"], pli["msg2"]["b"])
    s["g2"] = _vec8(pr["msg_bn2"]["g"], pli["msg_bn2"]["g"])
    s["bb2"] = _vec8(pr["msg_bn2"]["b"], pli["msg_bn2"]["b"])
    s["wp1"] = jnp.stack([pr["pos1"]["W"], pli["pos1"]["W"]])
    s["bp1"] = _vec8(pr["pos1"]["b"], pli["pos1"]["b"])
    s["gp"] = _vec8(pr["pos_bn"]["g"], pli["pos_bn"]["g"])
    s["bbp"] = _vec8(pr["pos_bn"]["b"], pli["pos_bn"]["b"])
    s["wp2"] = _vec8(pr["pos2"]["W"][:, 0], pli["pos2"]["W"][:, 0])
    bp2 = jnp.stack([pr["pos2"]["b"], pli["pos2"]["b"]])
    s["bp2"] = jnp.zeros((2, 8, 8), F32).at[:, 0, 0].set(bp2[:, 0])
    wur, wul = pr["upd1"]["W"], pli["upd1"]["W"]
    s["wua"] = jnp.stack([wur[:e], wul[:e]])
    s["wub"] = jnp.stack([wur[e:], wul[e:]])
    s["bu1"] = _vec8(pr["upd1"]["b"], pli["upd1"]["b"])
    s["gu1"] = _vec8(pr["upd_bn1"]["g"], pli["upd_bn1"]["g"])
    s["bu1b"] = _vec8(pr["upd_bn1"]["b"], pli["upd_bn1"]["b"])
    s["wu2"] = jnp.stack([pr["upd2"]["W"], pli["upd2"]["W"]])
    s["bu2"] = _vec8(pr["upd2"]["b"], pli["upd2"]["b"])
    s["gu2"] = _vec8(pr["upd_bn2"]["g"], pli["upd_bn2"]["g"])
    s["bu2b"] = _vec8(pr["upd_bn2"]["b"], pli["upd_bn2"]["b"])
    s["wo"] = jnp.stack([pr["out"]["W"], pli["out"]["W"]])
    s["bo"] = _vec8(pr["out"]["b"], pli["out"]["b"])
    return s


def kernel(rec_x, rec_pos, rec_edge_index, lig_x, lig_pos, lig_edge_index,
           params):
    x = jnp.concatenate([rec_x, lig_x], axis=0).reshape(NT, 1)
    pos = jnp.concatenate(
        [jnp.concatenate([rec_pos, lig_pos], axis=0),
         jnp.zeros((NT, 5), F32)], axis=1)

    dst = jnp.concatenate([rec_edge_index[1], lig_edge_index[1] + N])
    src = jnp.concatenate([rec_edge_index[0], lig_edge_index[0] + N])
    gidx = jnp.stack([dst, src]).reshape(2, NCHUNKS, CHUNK)
    sidx = jnp.stack([rec_edge_index[1], lig_edge_index[1]]).reshape(
        2, NCHUNKS_G, CHUNK)

    win_s = _vec8(params["lin_in_rec"]["W"][0], params["lin_in_lig"]["W"][0])
    bin_s = _vec8(params["lin_in_rec"]["b"], params["lin_in_lig"]["b"])
    h = _embed(x, win_s, bin_s)

    for li, (e, eo) in enumerate(_DIMS):
        lp = _stack_layer(params["rec_layers"][li], params["lig_layers"][li],
                          e)
        last = li == len(_DIMS) - 1
        off = 0 if last else e
        ngr = -(-(off + 4) // 128)

        Tdst, Tsrc = _node_pre(h, pos, lp["w1a"], lp["w1b"], lp["b1"], e)
        GA, GB = _sc_gather(Tdst, Tsrc, gidx, _tw(e))
        Y1, PD, st1 = _edge_t1(GA, GB, lp["w1c"], e)
        Y2, st2 = _bn_mm(Y1, st1, lp["g1"], lp["bb1"], lp["w2"], lp["b2"],
                         e, e, R_E, HB_E, E1)
        Y3, st3 = _bn_mm(Y2, st2, lp["g2"], lp["bb2"], lp["wp1"],
                         lp["bp1"], e, e, R_E, HB_E, E1)[:2]
        if last:
            groups = _edge_t4(Y3, st3, lp["gp"], lp["bbp"], lp["wp2"],
                              lp["bp2"], PD, e, last)
        else:
            groups = _edge_t4(Y3, st3, lp["gp"], lp["bbp"], lp["wp2"],
                              lp["bp2"], PD, e, last,
                              Y2, st2, lp["g2"], lp["bb2"])
        if not isinstance(groups, (tuple, list)):
            groups = [groups]

        zeros128 = jnp.zeros((NP_, 128), F32)
        Gsum = _sc_scatter(list(groups), sidx, zeros128, ngr)

        def _unpad(arr):
            return jnp.concatenate([arr[:N], arr[NP_:NP_ + N]], axis=0)

        Psum = _unpad(Gsum[off // 128])[:, off % 128:off % 128 + 8]
        if last:
            pos = _aggr_finish(Psum, pos)[0]
        else:
            Msum = _unpad(Gsum[0])[:, :e]
            pos, Mavg = _aggr_finish(Psum, pos, Msum)
            U1, stu1 = _nu1(h, Mavg, lp["wua"], lp["wub"], lp["bu1"], e)
            U2, stu2 = _bn_mm(U1, stu1, lp["gu1"], lp["bu1b"], lp["wu2"],
                              lp["bu2"], e, e, R_N, HB_N, N)
            h = _bn_mm(U2, stu2, lp["gu2"], lp["bu2b"], lp["wo"],
                       lp["bo"], e, eo, R_N, HB_N, N, emit_stats=False)[0]

    return (pos[:N, :3], pos[N:, :3])
